# Initial kernel scaffold; baseline (speedup 1.0000x reference)
import jax
import jax.numpy as jnp
from jax.experimental import pallas as pl


def kernel(x, edge_index, edge_type, batch, emb, W_rel1, W_root1, b1, W_rel2, W_root2, b2, W_lin, b_lin):
    raise NotImplementedError("write your pallas kernel here")


# speedup vs baseline: 1.9092x; 1.9092x over previous
import jax
import jax.numpy as jnp
from jax.experimental import pallas as pl

_R = 3
_B = 512
_BLK = 512


def _embed_body(x_ref, emb_ref, h_ref):
    x = x_ref[...]
    v = emb_ref.shape[0]
    oh = (x == jax.lax.broadcasted_iota(jnp.int32, (x.shape[0], v), 1)).astype(jnp.float32)
    h_ref[...] = jnp.dot(oh, emb_ref[...], preferred_element_type=jnp.float32)


def _embed(x, emb):
    n = x.shape[0]
    d = emb.shape[1]
    return pl.pallas_call(
        _embed_body,
        grid=(pl.cdiv(n, _BLK),),
        in_specs=[
            pl.BlockSpec((_BLK, 1), lambda i: (i, 0)),
            pl.BlockSpec(emb.shape, lambda i: (0, 0)),
        ],
        out_specs=pl.BlockSpec((_BLK, d), lambda i: (i, 0)),
        out_shape=jax.ShapeDtypeStruct((n, d), jnp.float32),
    )(x.reshape(n, 1).astype(jnp.int32), emb)


def _layer_body(h_ref, s_ref, c_ref, wroot_ref, b_ref, wrel_ref, out_ref):
    h = h_ref[...]
    din = h.shape[1]
    out = jnp.dot(h, wroot_ref[...], preferred_element_type=jnp.float32) + b_ref[...]
    s = s_ref[...]
    c = c_ref[...]
    for r in range(_R):
        mean = s[:, r * din:(r + 1) * din] / jnp.clip(c[:, r:r + 1], 1.0)
        out = out + jnp.dot(mean, wrel_ref[r], preferred_element_type=jnp.float32)
    out_ref[...] = jnp.maximum(out, 0.0)


def _layer(h, seg, cnt, w_root, b, w_rel):
    n, din = h.shape
    hdim = w_root.shape[1]
    return pl.pallas_call(
        _layer_body,
        grid=(pl.cdiv(n, _BLK),),
        in_specs=[
            pl.BlockSpec((_BLK, din), lambda i: (i, 0)),
            pl.BlockSpec((_BLK, _R * din), lambda i: (i, 0)),
            pl.BlockSpec((_BLK, _R), lambda i: (i, 0)),
            pl.BlockSpec((din, hdim), lambda i: (0, 0)),
            pl.BlockSpec((1, hdim), lambda i: (0, 0)),
            pl.BlockSpec((_R, din, hdim), lambda i: (0, 0, 0)),
        ],
        out_specs=pl.BlockSpec((_BLK, hdim), lambda i: (i, 0)),
        out_shape=jax.ShapeDtypeStruct((n, hdim), jnp.float32),
    )(h, seg.reshape(n, _R * din), cnt, w_root, b.reshape(1, hdim), w_rel)


def _final_body(s_ref, c_ref, wlin_ref, blin_ref, out_ref):
    pooled = s_ref[...] / jnp.clip(c_ref[...], 1.0)
    out_ref[...] = jnp.dot(pooled, wlin_ref[...], preferred_element_type=jnp.float32) + blin_ref[...]


def _final(seg, cnt, w_lin, b_lin):
    c = w_lin.shape[1]
    return pl.pallas_call(
        _final_body,
        out_shape=jax.ShapeDtypeStruct((_B, c), jnp.float32),
    )(seg, cnt.reshape(_B, 1), w_lin, b_lin.reshape(1, c))


def kernel(x, edge_index, edge_type, batch, emb, W_rel1, W_root1, b1,
           W_rel2, W_root2, b2, W_lin, b_lin):
    n = x.shape[0]
    src = edge_index[0]
    dst = edge_index[1]
    key = dst.astype(jnp.int32) * _R + edge_type.astype(jnp.int32)
    ones_e = jnp.ones(src.shape, jnp.float32)
    cnt = jax.ops.segment_sum(ones_e, key, num_segments=n * _R).reshape(n, _R)

    h = _embed(x, emb)
    s1 = jax.ops.segment_sum(h[src], key, num_segments=n * _R)
    h = _layer(h, s1, cnt, W_root1, b1, W_rel1)
    s2 = jax.ops.segment_sum(h[src], key, num_segments=n * _R)
    h = _layer(h, s2, cnt, W_root2, b2, W_rel2)

    sb = jax.ops.segment_sum(h, batch, num_segments=_B)
    cb = jax.ops.segment_sum(jnp.ones((n,), jnp.float32), batch, num_segments=_B)
    return _final(sb, cb, W_lin, b_lin)
